# Initial kernel scaffold; baseline (speedup 1.0000x reference)
import jax
import jax.numpy as jnp
from jax.experimental import pallas as pl


def kernel(X_dict, H_node_idx, H_edge_idx, H_values, Dv_inv_sqrt, De_inv, node_mask, W, b):
    raise NotImplementedError("write your pallas kernel here")


# speedup vs baseline: 2.7498x; 2.7498x over previous
import jax
import jax.numpy as jnp
from jax import lax
from jax.experimental import pallas as pl
from jax.experimental.pallas import tpu as pltpu
from jax.experimental.pallas import tpu_sc as plsc

_NC = 2
_NS = 16
_NW = _NC * _NS
_L = 16
_D = 128
_CHUNK = 128


def _project_kernel(x_ref, w_ref, b_ref, dvm_ref, o_ref):
    acc = jnp.dot(x_ref[...], w_ref[...], preferred_element_type=jnp.float32)
    acc = jnp.maximum(acc + b_ref[...], 0.0)
    o_ref[...] = acc * dvm_ref[...]


def _combine_kernel(p_ref, s_ref, o_ref):
    o_ref[...] = (p_ref[0] + p_ref[1]) * s_ref[...]


def _scale_kernel(p_ref, s_ref, o_ref):
    o_ref[...] = p_ref[...] * s_ref[...]


def _splat(vvec, j):
    return lax.gather(
        vvec, jnp.full((_L, 1), j, jnp.int32),
        lax.GatherDimensionNumbers(
            offset_dims=(), collapsed_slice_dims=(0,), start_index_map=(0,)),
        (1,), mode=lax.GatherScatterMode.PROMISE_IN_BOUNDS)


def _scale_rows(rows_v, vals_v):
    def scale(g, c2):
        vvec = vals_v[pl.ds(g * _L, _L)]
        for j in range(_L):
            v = _splat(vvec, j)
            r = g * _L + j
            for c in range(_D // _L):
                rows_v[r, pl.ds(c * _L, _L)] = rows_v[r, pl.ds(c * _L, _L)] * v
        return c2
    lax.fori_loop(0, _CHUNK // _L, scale, 0)


def _zero_acc(zero_v, acc_sh, rpt, sid):
    def zrow(r, carry):
        for c in range(_D // _L):
            zero_v[r, pl.ds(c * _L, _L)] = jnp.zeros((_L,), jnp.float32)
        return carry
    lax.fori_loop(0, rpt, zrow, 0)
    pltpu.sync_copy(zero_v, acc_sh.at[pl.ds(sid * rpt, rpt)])
    plsc.subcore_barrier()


def _sc_edge_pass(n_edges, n_chunks, per_tile):
    n_pad = -(-n_edges // (_NS * 8)) * (_NS * 8)
    rpt = n_pad // _NS
    mesh = plsc.VectorSubcoreMesh(core_axis_name="c", subcore_axis_name="s")

    def body(table_hbm, gidx_hbm, sidx_hbm, vals_hbm, out_hbm,
             gidx_v, sidx_v, vals_v, rows_v, zero_v, acc_sh, sem):
        cid = lax.axis_index("c")
        sid = lax.axis_index("s")
        wid = cid * _NS + sid
        _zero_acc(zero_v, acc_sh, rpt, sid)

        base = wid * per_tile

        def chunk(i, carry):
            off = base + i * _CHUNK
            pltpu.sync_copy(gidx_hbm.at[pl.ds(off, _CHUNK)], gidx_v)
            pltpu.sync_copy(sidx_hbm.at[pl.ds(off, _CHUNK)], sidx_v)
            pltpu.sync_copy(vals_hbm.at[pl.ds(off, _CHUNK)], vals_v)
            pltpu.async_copy(table_hbm.at[gidx_v], rows_v, sem).wait()
            _scale_rows(rows_v, vals_v)
            pltpu.sync_copy(rows_v, acc_sh.at[sidx_v], add=True)
            return carry
        lax.fori_loop(0, n_chunks, chunk, 0)

        plsc.subcore_barrier()
        pltpu.sync_copy(acc_sh.at[pl.ds(sid * rpt, rpt)],
                        out_hbm.at[cid, pl.ds(sid * rpt, rpt)])

    return pl.kernel(
        body,
        out_type=jax.ShapeDtypeStruct((_NC, n_pad, _D), jnp.float32),
        mesh=mesh,
        scratch_types=[
            pltpu.VMEM((_CHUNK,), jnp.int32),
            pltpu.VMEM((_CHUNK,), jnp.int32),
            pltpu.VMEM((_CHUNK,), jnp.float32),
            pltpu.VMEM((_CHUNK, _D), jnp.float32),
            pltpu.VMEM((rpt, _D), jnp.float32),
            pltpu.VMEM_SHARED((n_pad, _D), jnp.float32),
            pltpu.SemaphoreType.DMA,
        ],
    )


def _sc_node_pass(n_nodes, n_chunks, per_tile):
    n_pad = -(-n_nodes // (_NC * _NS * 8)) * (_NC * _NS * 8)
    n_half = n_pad // _NC
    rpt = n_half // _NS
    mesh = plsc.VectorSubcoreMesh(core_axis_name="c", subcore_axis_name="s")

    def body(table_hbm, gidx_hbm, sidx_hbm, vals_hbm, out_hbm,
             gidx_v, sidx_v, vals_v, rows_v, zero_v, acc_sh, sem):
        cid = lax.axis_index("c")
        sid = lax.axis_index("s")
        _zero_acc(zero_v, acc_sh, rpt, sid)

        base = sid * per_tile
        lo = cid * n_half

        def chunk(i, carry):
            off = base + i * _CHUNK
            pltpu.sync_copy(gidx_hbm.at[pl.ds(off, _CHUNK)], gidx_v)
            pltpu.sync_copy(sidx_hbm.at[pl.ds(off, _CHUNK)], sidx_v)
            pltpu.sync_copy(vals_hbm.at[pl.ds(off, _CHUNK)], vals_v)
            for g in range(_CHUNK // _L):
                v = sidx_v[pl.ds(g * _L, _L)]
                local = v - lo
                ok = (local >= 0) & (local < n_half)
                sidx_v[pl.ds(g * _L, _L)] = jnp.where(
                    ok, local, jnp.full((_L,), n_half, jnp.int32))
            pltpu.async_copy(table_hbm.at[gidx_v], rows_v, sem).wait()
            _scale_rows(rows_v, vals_v)
            pltpu.sync_copy(rows_v, acc_sh.at[sidx_v], add=True)
            return carry
        lax.fori_loop(0, n_chunks, chunk, 0)

        plsc.subcore_barrier()
        pltpu.sync_copy(acc_sh.at[pl.ds(sid * rpt, rpt)],
                        out_hbm.at[pl.ds(lo + sid * rpt, rpt)])

    return pl.kernel(
        body,
        out_type=jax.ShapeDtypeStruct((n_pad, _D), jnp.float32),
        mesh=mesh,
        scratch_types=[
            pltpu.VMEM((_CHUNK,), jnp.int32),
            pltpu.VMEM((_CHUNK,), jnp.int32),
            pltpu.VMEM((_CHUNK,), jnp.float32),
            pltpu.VMEM((_CHUNK, _D), jnp.float32),
            pltpu.VMEM((rpt, _D), jnp.float32),
            pltpu.VMEM_SHARED((n_half + 8, _D), jnp.float32),
            pltpu.SemaphoreType.DMA,
        ],
    )


def kernel(X_dict, H_node_idx, H_edge_idx, H_values, Dv_inv_sqrt, De_inv,
           node_mask, W, b):
    n_nodes, d_in = X_dict.shape
    d_out = W.shape[1]
    n_edges = De_inv.shape[0]
    nnz = H_node_idx.shape[0]

    dvm = (Dv_inv_sqrt * node_mask.astype(jnp.float32))[:, None]

    blk = 1000
    xs = pl.pallas_call(
        _project_kernel,
        grid=(n_nodes // blk,),
        in_specs=[
            pl.BlockSpec((blk, d_in), lambda i: (i, 0)),
            pl.BlockSpec((d_in, d_out), lambda i: (0, 0)),
            pl.BlockSpec((1, d_out), lambda i: (0, 0)),
            pl.BlockSpec((blk, 1), lambda i: (i, 0)),
        ],
        out_specs=pl.BlockSpec((blk, d_out), lambda i: (i, 0)),
        out_shape=jax.ShapeDtypeStruct((n_nodes, d_out), jnp.float32),
    )(X_dict, W, b[None, :], dvm)

    per_tile = -(-nnz // (_NW * _CHUNK)) * _CHUNK
    pad = per_tile * _NW - nnz
    nidx = jnp.pad(H_node_idx, (0, pad))
    eidx = jnp.pad(H_edge_idx, (0, pad))
    vals = jnp.pad(H_values, (0, pad))
    n_chunks = per_tile // _CHUNK

    y_part = _sc_edge_pass(n_edges, n_chunks, per_tile)(xs, nidx, eidx, vals)

    eb = n_edges // 2
    y = pl.pallas_call(
        _combine_kernel,
        grid=(2,),
        in_specs=[
            pl.BlockSpec((2, eb, d_out), lambda i: (0, i, 0)),
            pl.BlockSpec((eb, 1), lambda i: (i, 0)),
        ],
        out_specs=pl.BlockSpec((eb, d_out), lambda i: (i, 0)),
        out_shape=jax.ShapeDtypeStruct((n_edges, d_out), jnp.float32),
    )(y_part, De_inv[:, None])

    z_part = _sc_node_pass(n_nodes, n_chunks * 2, per_tile * 2)(
        y, eidx, nidx, vals)

    nb = n_nodes // 10
    z = pl.pallas_call(
        _scale_kernel,
        grid=(10,),
        in_specs=[
            pl.BlockSpec((nb, d_out), lambda i: (i, 0)),
            pl.BlockSpec((nb, 1), lambda i: (i, 0)),
        ],
        out_specs=pl.BlockSpec((nb, d_out), lambda i: (i, 0)),
        out_shape=jax.ShapeDtypeStruct((n_nodes, d_out), jnp.float32),
    )(z_part, Dv_inv_sqrt[:, None])
    return z
